# baseline (speedup 1.0000x reference)
import functools

import jax
import jax.numpy as jnp
from jax import lax
from jax.experimental import pallas as pl
from jax.experimental.pallas import tpu as pltpu
from jax.experimental.pallas import tpu_sc as plsc

NC = 2
NS = 16
C = 128
PW = 32
NP = 2
L = 16


def _sc_aggregate(features, src3, dst3, npad, k_chunks):
    n_nodes = features.shape[0]
    nr = n_nodes // NS
    rows_per_tile = npad // NS
    nfull, rem = divmod(rows_per_tile, C)

    mesh = plsc.VectorSubcoreMesh(core_axis_name="c", subcore_axis_name="s",
                                  num_cores=NC, num_subcores=NS)

    @functools.partial(
        pl.kernel,
        mesh=mesh,
        compiler_params=pltpu.CompilerParams(use_tc_tiling_on_sc=False,
                                             needs_layout_passes=False),
        out_type=(
            jax.ShapeDtypeStruct((NC * NP, npad, PW), jnp.float32),
            jax.ShapeDtypeStruct((NS, npad), jnp.float32),
        ),
        scratch_types=dict(
            src_v=pltpu.VMEM((k_chunks + 2, C), jnp.int32),
            dst_v=pltpu.VMEM((k_chunks, C), jnp.int32),
            rows0=pltpu.VMEM((C, PW), jnp.float32),
            rows1=pltpu.VMEM((C, PW), jnp.float32),
            zsum_v=pltpu.VMEM((C, PW), jnp.float32),
            hist_v=pltpu.VMEM((npad,), jnp.float32),
            gsems=(pltpu.SemaphoreType.DMA, pltpu.SemaphoreType.DMA),
            ssems=(pltpu.SemaphoreType.DMA, pltpu.SemaphoreType.DMA),
            feat_sh=pltpu.VMEM_SHARED((npad, PW), jnp.float32),
            sums_sh=pltpu.VMEM_SHARED((npad, PW), jnp.float32),
        ),
    )
    def agg(feat_hbm, src_hbm, dst_hbm, psum_hbm, hh_hbm,
            src_v, dst_v, rows0, rows1, zsum_v, hist_v,
            gsems, ssems, feat_sh, sums_sh):
        bufs = (rows0, rows1)
        cid = lax.axis_index("c")
        sid = lax.axis_index("s")
        base = sid * rows_per_tile

        zv = jnp.zeros((L,), jnp.float32)
        ov = jnp.ones((L,), jnp.float32)

        def fill_row(i, _):
            for cc in range(PW // L):
                zsum_v[i, pl.ds(cc * L, L)] = zv
            return 0
        lax.fori_loop(0, C, fill_row, 0)

        def zero_hist(i, _):
            hist_v[pl.ds(i * L, L)] = zv
            return 0
        lax.fori_loop(0, npad // L, zero_hist, 0)

        pltpu.sync_copy(src_hbm.at[sid], src_v.at[pl.ds(0, k_chunks)])
        pltpu.sync_copy(dst_hbm.at[sid], dst_v)
        for r in range(2):
            for cc in range(C // 16):
                src_v[k_chunks + r, pl.ds(cc * 16, 16)] = (
                    jnp.zeros((16,), jnp.int32))

        for p in range(NP):
            g = cid * NP + p

            pltpu.sync_copy(
                feat_hbm.at[pl.ds(sid * nr, nr), pl.ds(g * PW, PW)],
                feat_sh.at[pl.ds(sid * nr, nr)])
            for j in range(nfull):
                pltpu.sync_copy(zsum_v, sums_sh.at[pl.ds(base + j * C, C)])
            if rem:
                pltpu.sync_copy(zsum_v.at[pl.ds(0, rem)],
                                sums_sh.at[pl.ds(base + nfull * C, rem)])
            plsc.subcore_barrier()

            for b in range(2):
                pltpu.async_copy(feat_sh.at[src_v.at[b]], bufs[b], gsems[b])

            def step(t, _):
                for b in range(2):
                    j = 2 * t + b
                    rv = bufs[b]
                    pltpu.make_async_copy(feat_sh.at[src_v.at[j]], rv,
                                          gsems[b]).wait()
                    pltpu.async_copy(rv, sums_sh.at[dst_v.at[j]], ssems[b],
                                     add=True)
                    if p == 0:
                        for m in range(C // L):
                            dv = dst_v[j, pl.ds(m * L, L)]
                            plsc.addupdate_scatter(hist_v, [dv], ov)
                    pltpu.make_async_copy(rv, sums_sh.at[dst_v.at[j]],
                                          ssems[b]).wait()
                    pltpu.async_copy(feat_sh.at[src_v.at[j + 2]], rv,
                                     gsems[b])
                return 0
            lax.fori_loop(0, k_chunks // 2, step, 0)
            for b in range(2):
                pltpu.make_async_copy(feat_sh.at[src_v.at[0]], bufs[b],
                                      gsems[b]).wait()
            plsc.subcore_barrier()

            pltpu.sync_copy(sums_sh.at[pl.ds(base, rows_per_tile)],
                            psum_hbm.at[g, pl.ds(base, rows_per_tile)])
            if p == 0:
                @pl.when(cid == 0)
                def _publish_hist():
                    pltpu.sync_copy(hist_v, hh_hbm.at[sid])

    return agg(features, src3, dst3)


def _combine_body(ps_ref, pc_ref, o_ref):
    cnt = jnp.sum(pc_ref[...], axis=1)
    inv = 1.0 / jnp.maximum(cnt, 1.0)[:, None]
    for gg in range(NC * NP):
        o_ref[:, gg * PW:(gg + 1) * PW] = ps_ref[gg] * inv


def kernel(features, edge_index):
    n_nodes, d_feat = features.shape
    n_edges = edge_index.shape[1]

    per_tile = -(-n_edges // (NS * 2 * C)) * 2 * C
    k_chunks = per_tile // C
    tot = per_tile * NS
    npad = -(-(n_nodes + 1) // (NS * 8)) * (NS * 8)

    src = edge_index[0]
    dst = edge_index[1]
    pad = tot - n_edges
    if pad:
        src = jnp.concatenate([src, jnp.zeros((pad,), jnp.int32)])
        dst = jnp.concatenate([dst, jnp.full((pad,), n_nodes, jnp.int32)])
    src3 = src.reshape(NS, k_chunks, C)
    dst3 = dst.reshape(NS, k_chunks, C)

    psums, pcnts = _sc_aggregate(features, src3, dst3, npad, k_chunks)

    rblk = 2000
    grid = -(-n_nodes // rblk)
    out = pl.pallas_call(
        _combine_body,
        grid=(grid,),
        in_specs=[
            pl.BlockSpec((NC * NP, rblk, PW), lambda i: (0, i, 0)),
            pl.BlockSpec((rblk, NS), lambda i: (i, 0)),
        ],
        out_specs=pl.BlockSpec((rblk, d_feat), lambda i: (i, 0)),
        out_shape=jax.ShapeDtypeStruct((n_nodes, d_feat), jnp.float32),
    )(psums, pcnts.T)
    return out


# speedup vs baseline: 1.0678x; 1.0678x over previous
import functools

import jax
import jax.numpy as jnp
from jax import lax
from jax.experimental import pallas as pl
from jax.experimental.pallas import tpu as pltpu
from jax.experimental.pallas import tpu_sc as plsc

NC = 2
NS = 16
C = 128
PW = 32
NP = 2
L = 16


def _sc_aggregate(features, src3, dst3, npad, k_chunks):
    n_nodes = features.shape[0]
    nr = n_nodes // NS
    rows_per_tile = npad // NS
    nfull, rem = divmod(rows_per_tile, C)

    mesh = plsc.VectorSubcoreMesh(core_axis_name="c", subcore_axis_name="s",
                                  num_cores=NC, num_subcores=NS)

    @functools.partial(
        pl.kernel,
        mesh=mesh,
        compiler_params=pltpu.CompilerParams(use_tc_tiling_on_sc=False,
                                             needs_layout_passes=False),
        out_type=(
            jax.ShapeDtypeStruct((NC * NP, npad, PW), jnp.float32),
            jax.ShapeDtypeStruct((NS, npad), jnp.float32),
        ),
        scratch_types=dict(
            src_v=pltpu.VMEM((k_chunks + 2, C), jnp.int32),
            dst_v=pltpu.VMEM((k_chunks, C), jnp.int32),
            rows0=pltpu.VMEM((C, PW), jnp.float32),
            rows1=pltpu.VMEM((C, PW), jnp.float32),
            rows2=pltpu.VMEM((C, PW), jnp.float32),
            rows3=pltpu.VMEM((C, PW), jnp.float32),
            zsum_v=pltpu.VMEM((C, PW), jnp.float32),
            hist_v=pltpu.VMEM((npad,), jnp.float32),
            gsems=(pltpu.SemaphoreType.DMA, pltpu.SemaphoreType.DMA,
                   pltpu.SemaphoreType.DMA, pltpu.SemaphoreType.DMA),
            ssems=(pltpu.SemaphoreType.DMA, pltpu.SemaphoreType.DMA,
                   pltpu.SemaphoreType.DMA, pltpu.SemaphoreType.DMA),
            feat_sh=pltpu.VMEM_SHARED((npad, PW), jnp.float32),
            sums_sh=pltpu.VMEM_SHARED((npad, PW), jnp.float32),
        ),
    )
    def agg(feat_hbm, src_hbm, dst_hbm, psum_hbm, hh_hbm,
            src_v, dst_v, rows0, rows1, rows2, rows3, zsum_v, hist_v,
            gsems, ssems, feat_sh, sums_sh):
        bufs = (rows0, rows1, rows2, rows3)
        cid = lax.axis_index("c")
        sid = lax.axis_index("s")
        base = sid * rows_per_tile

        zv = jnp.zeros((L,), jnp.float32)
        ov = jnp.ones((L,), jnp.float32)

        def fill_row(i, _):
            for cc in range(PW // L):
                zsum_v[i, pl.ds(cc * L, L)] = zv
            return 0
        lax.fori_loop(0, C, fill_row, 0)

        def zero_hist(i, _):
            hist_v[pl.ds(i * L, L)] = zv
            return 0
        lax.fori_loop(0, npad // L, zero_hist, 0)

        pltpu.sync_copy(src_hbm.at[sid], src_v.at[pl.ds(0, k_chunks)])
        pltpu.sync_copy(dst_hbm.at[sid], dst_v)
        for r in range(2):
            for cc in range(C // 16):
                src_v[k_chunks + r, pl.ds(cc * 16, 16)] = (
                    jnp.zeros((16,), jnp.int32))

        for p in range(NP):
            g = cid * NP + p

            pltpu.sync_copy(
                feat_hbm.at[pl.ds(sid * nr, nr), pl.ds(g * PW, PW)],
                feat_sh.at[pl.ds(sid * nr, nr)])
            for j in range(nfull):
                pltpu.sync_copy(zsum_v, sums_sh.at[pl.ds(base + j * C, C)])
            if rem:
                pltpu.sync_copy(zsum_v.at[pl.ds(0, rem)],
                                sums_sh.at[pl.ds(base + nfull * C, rem)])
            plsc.subcore_barrier()

            for b in range(2):
                pltpu.async_copy(feat_sh.at[src_v.at[b]], bufs[b], gsems[b])

            def step(t, _):
                for b in range(4):
                    j = 4 * t + b
                    rv = bufs[b]
                    bn = (b + 2) % 4
                    pltpu.make_async_copy(feat_sh.at[src_v.at[j]], rv,
                                          gsems[b]).wait()
                    pltpu.async_copy(rv, sums_sh.at[dst_v.at[j]], ssems[b],
                                     add=True)
                    if p == 0:
                        for m in range(C // L):
                            dv = dst_v[j, pl.ds(m * L, L)]
                            plsc.addupdate_scatter(hist_v, [dv], ov)

                    def _prefetch():
                        pltpu.make_async_copy(bufs[bn],
                                              sums_sh.at[dst_v.at[j]],
                                              ssems[bn]).wait()
                        pltpu.async_copy(feat_sh.at[src_v.at[j + 2]],
                                         bufs[bn], gsems[bn])
                    if b < 2:
                        @pl.when(t > 0)
                        def _pf_guard():
                            _prefetch()

                        @pl.when(t == 0)
                        def _pf_first():
                            pltpu.async_copy(feat_sh.at[src_v.at[j + 2]],
                                             bufs[bn], gsems[bn])
                    else:
                        _prefetch()
                return 0
            lax.fori_loop(0, k_chunks // 4, step, 0)
            for b in range(2):
                pltpu.make_async_copy(bufs[b + 2],
                                      sums_sh.at[dst_v.at[0]],
                                      ssems[b + 2]).wait()
                pltpu.make_async_copy(feat_sh.at[src_v.at[0]], bufs[b],
                                      gsems[b]).wait()
            plsc.subcore_barrier()

            pltpu.sync_copy(sums_sh.at[pl.ds(base, rows_per_tile)],
                            psum_hbm.at[g, pl.ds(base, rows_per_tile)])
            if p == 0:
                @pl.when(cid == 0)
                def _publish_hist():
                    pltpu.sync_copy(hist_v, hh_hbm.at[sid])

    return agg(features, src3, dst3)


def _combine_body(ps_ref, pc_ref, o_ref):
    cnt = jnp.sum(pc_ref[...], axis=1)
    inv = 1.0 / jnp.maximum(cnt, 1.0)[:, None]
    for gg in range(NC * NP):
        o_ref[:, gg * PW:(gg + 1) * PW] = ps_ref[gg] * inv


def kernel(features, edge_index):
    n_nodes, d_feat = features.shape
    n_edges = edge_index.shape[1]

    per_tile = -(-n_edges // (NS * 4 * C)) * 4 * C
    k_chunks = per_tile // C
    tot = per_tile * NS
    npad = -(-(n_nodes + 1) // (NS * 8)) * (NS * 8)

    src = edge_index[0]
    dst = edge_index[1]
    pad = tot - n_edges
    if pad:
        src = jnp.concatenate([src, jnp.zeros((pad,), jnp.int32)])
        dst = jnp.concatenate([dst, jnp.full((pad,), n_nodes, jnp.int32)])
    src3 = src.reshape(NS, k_chunks, C)
    dst3 = dst.reshape(NS, k_chunks, C)

    psums, pcnts = _sc_aggregate(features, src3, dst3, npad, k_chunks)

    rblk = 2000
    grid = -(-n_nodes // rblk)
    out = pl.pallas_call(
        _combine_body,
        grid=(grid,),
        in_specs=[
            pl.BlockSpec((NC * NP, rblk, PW), lambda i: (0, i, 0)),
            pl.BlockSpec((rblk, NS), lambda i: (i, 0)),
        ],
        out_specs=pl.BlockSpec((rblk, d_feat), lambda i: (i, 0)),
        out_shape=jax.ShapeDtypeStruct((n_nodes, d_feat), jnp.float32),
    )(psums, pcnts.T)
    return out
